# Initial kernel scaffold; baseline (speedup 1.0000x reference)
import jax
import jax.numpy as jnp
from jax.experimental import pallas as pl


def kernel(feat, member_idx, cluster_mask, pe_idx, global_attn, pre_table, W_q, b_q, W_kv, b_kv, blank_k, blank_v, W_pe, b_pe, W_proj, b_proj):
    raise NotImplementedError("write your pallas kernel here")


# speedup vs baseline: 9.5073x; 9.5073x over previous
import functools

import jax
import jax.numpy as jnp
from jax import lax
from jax.experimental import pallas as pl
from jax.experimental.pallas import tpu as pltpu
from jax.experimental.pallas import tpu_sc as plsc

_NC = 2
_NS = 16
_NW = _NC * _NS


def _proj_body(x_ref, wq_ref, bq_ref, wk_ref, bk_ref, wv_ref, bv_ref, scale_ref,
               q_ref, k_ref, v_ref):
    x = x_ref[...]
    q_ref[...] = (jnp.dot(x, wq_ref[...]) + bq_ref[...]) * scale_ref[0, 0]
    k_ref[...] = jnp.dot(x, wk_ref[...]) + bk_ref[...]
    v_ref[...] = jnp.dot(x, wv_ref[...]) + bv_ref[...]


def _attn_body(q_ref, kg_ref, vg_ref, peg_ref, mask_ref, lg_ref, s_ref, srep_ref,
               wpe_ref, bpe_ref, blankk_ref, blankv_ref, wproj_ref, bproj_ref,
               out_ref, *, tb, m):
    c = q_ref.shape[-1]
    q = q_ref[...]
    s_sel = s_ref[...]
    kg = kg_ref[...]
    qe = jnp.broadcast_to(q[:, None, :], (tb, m, c)).reshape(tb * m, c)
    scores = jnp.dot(qe * kg, s_sel)
    pe = jnp.dot(peg_ref[...], wpe_ref[...]) + bpe_ref[...]
    lg = lg_ref[0, 0]
    s3 = scores.reshape(tb, m, 8) + pe.reshape(tb, m, 8)
    s3 = s3 + ((1.0 - mask_ref[...]) * (-100.0) * lg)[:, :, None]
    bl = jnp.clip(jnp.dot(q * blankk_ref[...], s_sel), -5.0, 5.0)
    mx = jnp.maximum(jnp.max(s3, axis=1), bl)
    e3 = jnp.exp(s3 - mx[:, None, :])
    eb = jnp.exp(bl - mx)
    den = jnp.sum(e3, axis=1) + eb
    attn = (e3 / den[:, None, :]).reshape(tb * m, 8)
    ar = jnp.dot(attn, srep_ref[...])
    out = jnp.sum((ar * vg_ref[...]).reshape(tb, m, c), axis=1)
    out = out + jnp.dot(eb / den, srep_ref[...]) * blankv_ref[...]
    out_ref[...] = jnp.dot(out, wproj_ref[...]) + bproj_ref[...]


def _make_sc_gather(rows, c, pw, chk, per_w):
    n_chunks = per_w // chk
    mesh = plsc.VectorSubcoreMesh(core_axis_name="c", subcore_axis_name="s")

    @functools.partial(
        pl.kernel,
        mesh=mesh,
        out_type=[
            jax.ShapeDtypeStruct((rows, c), jnp.float32),
            jax.ShapeDtypeStruct((rows, c), jnp.float32),
            jax.ShapeDtypeStruct((rows, pw), jnp.float32),
        ],
        scratch_types=[
            pltpu.VMEM((chk,), jnp.int32),
            pltpu.VMEM((chk,), jnp.int32),
            pltpu.VMEM((chk, c), jnp.float32),
            pltpu.VMEM((chk, c), jnp.float32),
            pltpu.VMEM((chk, pw), jnp.float32),
            pltpu.SemaphoreType.DMA,
            pltpu.SemaphoreType.DMA,
            pltpu.SemaphoreType.DMA,
        ],
        compiler_params=pltpu.CompilerParams(use_tc_tiling_on_sc=False),
    )
    def sc_gather(gidx_hbm, pidx_hbm, k_hbm, v_hbm, pre_hbm,
                  kg_hbm, vg_hbm, peg_hbm,
                  idx_v, pidx_v, kbuf, vbuf, pbuf, semk, semv, semp):
        wid = lax.axis_index("s") * _NC + lax.axis_index("c")
        w0 = wid * per_w

        def body(i, carry):
            base = w0 + i * chk
            pltpu.sync_copy(gidx_hbm.at[pl.ds(base, chk)], idx_v)
            pltpu.sync_copy(pidx_hbm.at[pl.ds(base, chk)], pidx_v)
            ck = pltpu.async_copy(k_hbm.at[idx_v], kbuf, semk)
            cv = pltpu.async_copy(v_hbm.at[idx_v], vbuf, semv)
            cp = pltpu.async_copy(pre_hbm.at[pidx_v], pbuf, semp)
            ck.wait()
            cv.wait()
            cp.wait()
            pltpu.sync_copy(kbuf, kg_hbm.at[pl.ds(base, chk)])
            pltpu.sync_copy(vbuf, vg_hbm.at[pl.ds(base, chk)])
            pltpu.sync_copy(pbuf, peg_hbm.at[pl.ds(base, chk)])
            return carry

        lax.fori_loop(0, n_chunks, body, 0)

    return sc_gather


def kernel(feat, member_idx, cluster_mask, pe_idx, global_attn, pre_table,
           W_q, b_q, W_kv, b_kv, blank_k, blank_v, W_pe, b_pe, W_proj, b_proj):
    B, N, C = feat.shape
    M = member_idx.shape[-1]
    H = W_pe.shape[1]
    CH = C // H
    T = pre_table.shape[0]
    BN = B * N
    R = BN * M
    scale = jnp.float32(CH) ** -0.5

    f32 = jnp.float32
    x = feat.reshape(BN, C)

    hcol = jnp.arange(C)
    h_of = hcol // CH
    c_of = hcol % CH
    kcols = h_of * (2 * CH) + c_of
    vcols = kcols + CH
    Wk = W_kv[:, kcols]
    Wv = W_kv[:, vcols]
    bk = b_kv[kcols].reshape(1, C)
    bv = b_kv[vcols].reshape(1, C)
    bq2 = b_q.reshape(1, C)
    scale_arr = jnp.full((1, 1), scale, f32)

    TB1 = 256
    g1 = BN // TB1
    q2, k2, v2 = pl.pallas_call(
        _proj_body,
        grid=(g1,),
        in_specs=[
            pl.BlockSpec((TB1, C), lambda i: (i, 0)),
            pl.BlockSpec((C, C), lambda i: (0, 0)),
            pl.BlockSpec((1, C), lambda i: (0, 0)),
            pl.BlockSpec((C, C), lambda i: (0, 0)),
            pl.BlockSpec((1, C), lambda i: (0, 0)),
            pl.BlockSpec((C, C), lambda i: (0, 0)),
            pl.BlockSpec((1, C), lambda i: (0, 0)),
            pl.BlockSpec((1, 1), lambda i: (0, 0), memory_space=pltpu.SMEM),
        ],
        out_specs=[
            pl.BlockSpec((TB1, C), lambda i: (i, 0)),
            pl.BlockSpec((TB1, C), lambda i: (i, 0)),
            pl.BlockSpec((TB1, C), lambda i: (i, 0)),
        ],
        out_shape=[jax.ShapeDtypeStruct((BN, C), f32)] * 3,
    )(x, W_q, bq2, Wk, bk, Wv, bv, scale_arr)

    gidx = (member_idx.astype(jnp.int32)
            + (jnp.arange(B, dtype=jnp.int32) * N)[:, None, None]).reshape(R)
    pidx = pe_idx.astype(jnp.int32).reshape(R)
    PW = 8
    pre8 = jnp.zeros((T, PW), f32).at[:, :5].set(pre_table)

    CHK = 128
    per_w = R // _NW
    sc_gather = _make_sc_gather(R, C, PW, CHK, per_w)
    kg, vg, peg = sc_gather(gidx, pidx, k2, v2, pre8)

    S = (h_of[:, None] == jnp.arange(8)[None, :]).astype(f32)
    Srep = S.T
    Wpe8 = jnp.zeros((PW, 8), f32).at[:5, :H].set(W_pe)
    bpe8 = jnp.zeros((1, 8), f32).at[0, :H].set(b_pe)
    lg = jnp.logical_not(global_attn).astype(f32).reshape(1, 1)
    mask2 = cluster_mask.reshape(BN, M)

    TB2 = 64
    g2 = BN // TB2
    out = pl.pallas_call(
        functools.partial(_attn_body, tb=TB2, m=M),
        grid=(g2,),
        in_specs=[
            pl.BlockSpec((TB2, C), lambda i: (i, 0)),
            pl.BlockSpec((TB2 * M, C), lambda i: (i, 0)),
            pl.BlockSpec((TB2 * M, C), lambda i: (i, 0)),
            pl.BlockSpec((TB2 * M, PW), lambda i: (i, 0)),
            pl.BlockSpec((TB2, M), lambda i: (i, 0)),
            pl.BlockSpec((1, 1), lambda i: (0, 0), memory_space=pltpu.SMEM),
            pl.BlockSpec((C, 8), lambda i: (0, 0)),
            pl.BlockSpec((8, C), lambda i: (0, 0)),
            pl.BlockSpec((PW, 8), lambda i: (0, 0)),
            pl.BlockSpec((1, 8), lambda i: (0, 0)),
            pl.BlockSpec((1, C), lambda i: (0, 0)),
            pl.BlockSpec((1, C), lambda i: (0, 0)),
            pl.BlockSpec((C, C), lambda i: (0, 0)),
            pl.BlockSpec((1, C), lambda i: (0, 0)),
        ],
        out_specs=pl.BlockSpec((TB2, C), lambda i: (i, 0)),
        out_shape=jax.ShapeDtypeStruct((BN, C), f32),
    )(q2, kg, vg, peg, mask2, lg, S, Srep, Wpe8, bpe8,
      blank_k.reshape(1, C), blank_v.reshape(1, C), W_proj, b_proj.reshape(1, C))

    return out.reshape(B, N, C)


# speedup vs baseline: 18.3415x; 1.9292x over previous
import functools

import jax
import jax.numpy as jnp
from jax import lax
from jax.experimental import pallas as pl
from jax.experimental.pallas import tpu as pltpu
from jax.experimental.pallas import tpu_sc as plsc

_NC = 2
_NS = 16
_NW = _NC * _NS


def _proj_body(x_ref, wq_ref, bq_ref, wkv_ref, bkv_ref, q_ref, kv_ref):
    x = x_ref[...]
    q_ref[...] = jnp.dot(x, wq_ref[...]) + bq_ref[...]
    kv_ref[...] = jnp.dot(x, wkv_ref[...]) + bkv_ref[...]


def _attn_body(q_ref, kvg_ref, peg_ref, mask_ref, lg_ref, s_ref, srepv_ref,
               psel_ref, srep_ref, wpe_ref, bpe_ref, blankk_ref, blankv_ref,
               wproj_ref, bproj_ref, out_ref, *, tb, m):
    c2 = kvg_ref.shape[-1]
    q = q_ref[...]
    s_sel = s_ref[...]
    kvg = kvg_ref[...]
    qe = jnp.broadcast_to(q[:, None, :], (tb, m, c2)).reshape(tb * m, c2)
    scores = jnp.dot(qe * kvg, s_sel)
    pe = jnp.dot(peg_ref[...], wpe_ref[...]) + bpe_ref[...]
    lg = lg_ref[0, 0]
    s3 = scores.reshape(tb, m, 8) + pe.reshape(tb, m, 8)
    s3 = s3 + ((1.0 - mask_ref[...]) * (-100.0) * lg)[:, :, None]
    bl = jnp.clip(jnp.dot(q * blankk_ref[...], s_sel), -5.0, 5.0)
    mx = jnp.maximum(jnp.max(s3, axis=1), bl)
    e3 = jnp.exp(s3 - mx[:, None, :])
    eb = jnp.exp(bl - mx)
    den = jnp.sum(e3, axis=1) + eb
    attn = (e3 / den[:, None, :]).reshape(tb * m, 8)
    ar = jnp.dot(attn, srepv_ref[...])
    o2 = jnp.sum((ar * kvg).reshape(tb, m, c2), axis=1)
    out = jnp.dot(o2, psel_ref[...])
    out = out + jnp.dot(eb / den, srep_ref[...]) * blankv_ref[...]
    out_ref[...] = jnp.dot(out, wproj_ref[...]) + bproj_ref[...]


def _make_sc_gather(rows, width, chk, per_w, nbuf, tc_tiling):
    n_chunks = per_w // chk
    t_steps = n_chunks // nbuf
    mesh = plsc.VectorSubcoreMesh(core_axis_name="c", subcore_axis_name="s")

    @functools.partial(
        pl.kernel,
        mesh=mesh,
        out_type=jax.ShapeDtypeStruct((rows, width), jnp.float32),
        scratch_types=(
            [pltpu.VMEM((per_w,), jnp.int32),
             pltpu.VMEM((nbuf * chk, width), jnp.float32)]
            + [pltpu.SemaphoreType.DMA] * (2 * nbuf)
        ),
        compiler_params=pltpu.CompilerParams(use_tc_tiling_on_sc=tc_tiling),
    )
    def sc_gather(idx_hbm, table_hbm, out_hbm, idx_all, bufs, *sems):
        semg = sems[:nbuf]
        semw = sems[nbuf:]
        wid = lax.axis_index("s") * _NC + lax.axis_index("c")
        w0 = wid * per_w
        pltpu.sync_copy(idx_hbm.at[pl.ds(w0, per_w)], idx_all)

        def gat(i, b):
            return pltpu.make_async_copy(
                table_hbm.at[idx_all.at[pl.ds(i * chk, chk)]],
                bufs.at[pl.ds(b * chk, chk)], semg[b])

        def sto(i, b):
            return pltpu.make_async_copy(
                bufs.at[pl.ds(b * chk, chk)],
                out_hbm.at[pl.ds(w0 + i * chk, chk)], semw[b])

        for b in range(nbuf):
            gat(b, b).start()

        def body(t, carry):
            for b in range(nbuf):
                i = t * nbuf + b
                gat(i, b).wait()
                sto(i, b).start()

                @pl.when(t < t_steps - 1)
                def _():
                    sto(i, b).wait()
                    gat(i + nbuf, b).start()

            return carry

        lax.fori_loop(0, t_steps, body, 0)
        for b in range(nbuf):
            sto((t_steps - 1) * nbuf + b, b).wait()

    return sc_gather


def kernel(feat, member_idx, cluster_mask, pe_idx, global_attn, pre_table,
           W_q, b_q, W_kv, b_kv, blank_k, blank_v, W_pe, b_pe, W_proj, b_proj):
    B, N, C = feat.shape
    M = member_idx.shape[-1]
    H = W_pe.shape[1]
    CH = C // H
    C2 = 2 * C
    T = pre_table.shape[0]
    BN = B * N
    R = BN * M
    scale = jnp.float32(CH) ** -0.5

    f32 = jnp.float32
    x = feat.reshape(BN, C)

    col = jnp.arange(C2)
    h_of = col // (2 * CH)
    is_k = (col % (2 * CH)) < CH
    c_of = col % CH
    hm_of = h_of * CH + c_of
    emb = jnp.zeros((C, C2), f32).at[hm_of, col].set(jnp.where(is_k, 1.0, 0.0))
    Wq_int = (W_q * scale) @ emb
    bq_int = ((b_q * scale) @ emb).reshape(1, C2)
    blankk_int = (blank_k @ emb).reshape(1, C2)

    TB1 = 256
    g1 = BN // TB1
    qint, kv2 = pl.pallas_call(
        _proj_body,
        grid=(g1,),
        in_specs=[
            pl.BlockSpec((TB1, C), lambda i: (i, 0)),
            pl.BlockSpec((C, C2), lambda i: (0, 0)),
            pl.BlockSpec((1, C2), lambda i: (0, 0)),
            pl.BlockSpec((C, C2), lambda i: (0, 0)),
            pl.BlockSpec((1, C2), lambda i: (0, 0)),
        ],
        out_specs=[
            pl.BlockSpec((TB1, C2), lambda i: (i, 0)),
            pl.BlockSpec((TB1, C2), lambda i: (i, 0)),
        ],
        out_shape=[jax.ShapeDtypeStruct((BN, C2), f32)] * 2,
    )(x, Wq_int, bq_int, W_kv, b_kv.reshape(1, C2))

    gidx = (member_idx.astype(jnp.int32)
            + (jnp.arange(B, dtype=jnp.int32) * N)[:, None, None]).reshape(R)
    pidx = pe_idx.astype(jnp.int32).reshape(R)
    PW = 8
    pre8 = jnp.zeros((T, PW), f32).at[:, :5].set(pre_table)

    per_w = R // _NW
    kvg = _make_sc_gather(R, C2, 64, per_w, 3, True)(gidx, kv2)
    peg = _make_sc_gather(R, PW, 128, per_w, 3, False)(pidx, pre8)

    h8 = jnp.arange(8)[None, :]
    S = ((h_of[:, None] == h8) & is_k[:, None]).astype(f32)
    SrepV = (((h_of[:, None] == h8) & (~is_k)[:, None]).astype(f32)).T
    Psel = jnp.zeros((C2, C), f32).at[col, hm_of].set(
        jnp.where(is_k, 0.0, 1.0))
    Srep = ((jnp.arange(C) // CH)[:, None] == h8).astype(f32).T
    Wpe8 = jnp.zeros((PW, 8), f32).at[:5, :H].set(W_pe)
    bpe8 = jnp.zeros((1, 8), f32).at[0, :H].set(b_pe)
    lg = jnp.logical_not(global_attn).astype(f32).reshape(1, 1)
    mask2 = cluster_mask.reshape(BN, M)

    TB2 = 64
    g2 = BN // TB2
    out = pl.pallas_call(
        functools.partial(_attn_body, tb=TB2, m=M),
        grid=(g2,),
        in_specs=[
            pl.BlockSpec((TB2, C2), lambda i: (i, 0)),
            pl.BlockSpec((TB2 * M, C2), lambda i: (i, 0)),
            pl.BlockSpec((TB2 * M, PW), lambda i: (i, 0)),
            pl.BlockSpec((TB2, M), lambda i: (i, 0)),
            pl.BlockSpec((1, 1), lambda i: (0, 0), memory_space=pltpu.SMEM),
            pl.BlockSpec((C2, 8), lambda i: (0, 0)),
            pl.BlockSpec((8, C2), lambda i: (0, 0)),
            pl.BlockSpec((C2, C), lambda i: (0, 0)),
            pl.BlockSpec((8, C), lambda i: (0, 0)),
            pl.BlockSpec((PW, 8), lambda i: (0, 0)),
            pl.BlockSpec((1, 8), lambda i: (0, 0)),
            pl.BlockSpec((1, C2), lambda i: (0, 0)),
            pl.BlockSpec((1, C), lambda i: (0, 0)),
            pl.BlockSpec((C, C), lambda i: (0, 0)),
            pl.BlockSpec((1, C), lambda i: (0, 0)),
        ],
        out_specs=pl.BlockSpec((TB2, C), lambda i: (i, 0)),
        out_shape=jax.ShapeDtypeStruct((BN, C), f32),
    )(qint, kvg, peg, mask2, lg, S, SrepV, Psel, Srep, Wpe8, bpe8,
      blankk_int, blank_v.reshape(1, C), W_proj, b_proj.reshape(1, C))

    return out.reshape(B, N, C)
